# baseline (speedup 1.0000x reference)
import jax
import jax.numpy as jnp
from jax import lax
from jax.experimental import pallas as pl
from jax.experimental.pallas import tpu as pltpu

_B = 1024
_DIM = 128
_R = 65536
_T = 0.1
_W = 1024
_NB = _R // _W
_GRID = _NB + 1


def _prep_body(q_ref, k_ref, qn_s_ref, knt_ext_ref, lpos_ref):
    q = q_ref[...]
    k = k_ref[...]
    qn = q / jnp.maximum(jnp.sqrt(jnp.sum(q * q, axis=1, keepdims=True)), 1e-12)
    kn = k / jnp.maximum(jnp.sqrt(jnp.sum(k * k, axis=1, keepdims=True)), 1e-12)
    lpos_ref[...] = jnp.sum(qn * kn, axis=1, keepdims=True) * (1.0 / _T)
    qn_s_ref[...] = (qn * (1.0 / _T)).astype(jnp.bfloat16)
    knt_ext_ref[...] = kn.T


def _main_body(ptr_ref, qn_s_ref, knt_ref, lpos_ref, qa_ref, qb_ref, idx_ref,
               qidx_ref, logits_ref, nq_ref, nqi_ref):
    b = pl.program_id(0)
    ptr = ptr_ref[0]

    qa = qa_ref[...].astype(jnp.bfloat16)
    qb_f32 = qb_ref[...]
    qb = qb_f32.astype(jnp.bfloat16)
    shifted = jnp.concatenate([qa[:, _W - 1:], qb[:, :_W - 1]], axis=1)
    logits_ref[...] = jnp.dot(qn_s_ref[...], shifted,
                              preferred_element_type=jnp.float32)

    @pl.when(b == 0)
    def _():
        logits_ref[:, 0:1] = lpos_ref[...]

    be = jnp.minimum(b, _NB - 1)
    c0 = be * _W
    cols = c0 + lax.broadcasted_iota(jnp.int32, (1, _W), 1)
    m = (cols >= ptr) & (cols < ptr + _B)
    off = jnp.clip(be - ptr // _W + 1, 0, 2) * _W
    nq_ref[...] = jnp.where(m, knt_ref[:, pl.ds(off, _W)], qb_f32)
    nqi_ref[...] = jnp.where(m[None], idx_ref[:, :, pl.ds(off, _W)],
                             qidx_ref[...])


def _prologue(q, k, interpret=False):
    return pl.pallas_call(
        _prep_body,
        out_shape=(
            jax.ShapeDtypeStruct((_B, _DIM), jnp.bfloat16),
            jax.ShapeDtypeStruct((_DIM, _B), jnp.float32),
            jax.ShapeDtypeStruct((_B, 1), jnp.float32),
        ),
        interpret=interpret,
    )(q, k)


def _main(ptr_arr, qn_s, knt_ext, lpos, queue, idx_ext, qidx_r, interpret=False):
    grid_spec = pltpu.PrefetchScalarGridSpec(
        num_scalar_prefetch=1,
        grid=(_GRID,),
        in_specs=[
            pl.BlockSpec((_B, _DIM), lambda b, s: (0, 0)),
            pl.BlockSpec((_DIM, 3 * _B), lambda b, s: (0, 0)),
            pl.BlockSpec((_B, 1), lambda b, s: (0, 0)),
            pl.BlockSpec((_DIM, _W), lambda b, s: (0, jnp.maximum(b - 1, 0))),
            pl.BlockSpec((_DIM, _W), lambda b, s: (0, jnp.minimum(b, _NB - 1))),
            pl.BlockSpec((1, 1, 3 * _B), lambda b, s: (0, 0, 0)),
            pl.BlockSpec((1, 1, _W), lambda b, s: (jnp.minimum(b, _NB - 1), 0, 0)),
        ],
        out_specs=[
            pl.BlockSpec((_B, _W), lambda b, s: (0, b)),
            pl.BlockSpec((_DIM, _W), lambda b, s: (0, jnp.minimum(b, _NB - 1))),
            pl.BlockSpec((1, 1, _W), lambda b, s: (jnp.minimum(b, _NB - 1), 0, 0)),
        ],
    )
    return pl.pallas_call(
        _main_body,
        grid_spec=grid_spec,
        out_shape=(
            jax.ShapeDtypeStruct((_B, _R + 1), jnp.float32),
            jax.ShapeDtypeStruct((_DIM, _R), jnp.float32),
            jax.ShapeDtypeStruct((_NB, 1, _W), jnp.int32),
        ),
        compiler_params=pltpu.CompilerParams(
            dimension_semantics=("parallel",),
        ),
        interpret=interpret,
    )(ptr_arr, qn_s, knt_ext, lpos, queue, queue, idx_ext, qidx_r)


def kernel(q, k, queue, index, queue_index, ptr, interpret=False):
    qn_s, knt, lpos = _prologue(q, k, interpret=interpret)
    ptr_c = jnp.clip(jnp.asarray(ptr, jnp.int32), 0, _R - _B)
    ptr_arr = ptr_c[None]
    rr = ptr_c % _W
    knt_ext = jnp.roll(jnp.pad(knt, ((0, 0), (_B, _B))), rr, axis=1)
    idx_ext = jnp.roll(jnp.pad(index, (_B, _B)), rr)[None, None, :]
    qidx_r = queue_index.reshape(_NB, 1, _W)
    logits, nq, nqi = _main(ptr_arr, qn_s, knt_ext, lpos, queue, idx_ext,
                            qidx_r, interpret=interpret)
    return logits, nq, nqi.reshape(_R)


# speedup vs baseline: 1.0191x; 1.0191x over previous
import jax
import jax.numpy as jnp
from jax import lax
from jax.experimental import pallas as pl
from jax.experimental.pallas import tpu as pltpu

_B = 1024
_DIM = 128
_R = 65536
_T = 0.1
_W = 2048
_NB = _R // _W
_GRID = _NB + 1


def _prep_body(q_ref, k_ref, qn_s_ref, knt_ext_ref, lpos_ref):
    q = q_ref[...]
    k = k_ref[...]
    qn = q / jnp.maximum(jnp.sqrt(jnp.sum(q * q, axis=1, keepdims=True)), 1e-12)
    kn = k / jnp.maximum(jnp.sqrt(jnp.sum(k * k, axis=1, keepdims=True)), 1e-12)
    lpos_ref[...] = jnp.sum(qn * kn, axis=1, keepdims=True) * (1.0 / _T)
    qn_s_ref[...] = (qn * (1.0 / _T)).astype(jnp.bfloat16)
    knt_ext_ref[...] = kn.T


def _main_body(ptr_ref, qn_s_ref, knt_ref, lpos_ref, qa_ref, qb_ref, idx_ref,
               qidx_ref, logits_ref, nq_ref, nqi_ref):
    b = pl.program_id(0)
    ptr = ptr_ref[0]

    qa = qa_ref[...].astype(jnp.bfloat16)
    qb_f32 = qb_ref[...]
    qb = qb_f32.astype(jnp.bfloat16)
    shifted = jnp.concatenate([qa[:, _W - 1:], qb[:, :_W - 1]], axis=1)
    logits_ref[...] = jnp.dot(qn_s_ref[...], shifted,
                              preferred_element_type=jnp.float32)

    @pl.when(b == 0)
    def _():
        logits_ref[:, 0:1] = lpos_ref[...]

    be = jnp.minimum(b, _NB - 1)
    c0 = be * _W
    cols = c0 + lax.broadcasted_iota(jnp.int32, (1, _W), 1)
    m = (cols >= ptr) & (cols < ptr + _B)
    off = jnp.clip(be - ptr // _W + 1, 0, 2) * _W
    nq_ref[...] = jnp.where(m, knt_ref[:, pl.ds(off, _W)], qb_f32)
    nqi_ref[...] = jnp.where(m[None], idx_ref[:, :, pl.ds(off, _W)],
                             qidx_ref[...])


def _prologue(q, k, interpret=False):
    return pl.pallas_call(
        _prep_body,
        out_shape=(
            jax.ShapeDtypeStruct((_B, _DIM), jnp.bfloat16),
            jax.ShapeDtypeStruct((_DIM, _B), jnp.float32),
            jax.ShapeDtypeStruct((_B, 1), jnp.float32),
        ),
        interpret=interpret,
    )(q, k)


def _main(ptr_arr, qn_s, knt_ext, lpos, queue, idx_ext, qidx_r, interpret=False):
    grid_spec = pltpu.PrefetchScalarGridSpec(
        num_scalar_prefetch=1,
        grid=(_GRID,),
        in_specs=[
            pl.BlockSpec((_B, _DIM), lambda b, s: (0, 0)),
            pl.BlockSpec((_DIM, 3 * _W), lambda b, s: (0, 0)),
            pl.BlockSpec((_B, 1), lambda b, s: (0, 0)),
            pl.BlockSpec((_DIM, _W), lambda b, s: (0, jnp.maximum(b - 1, 0))),
            pl.BlockSpec((_DIM, _W), lambda b, s: (0, jnp.minimum(b, _NB - 1))),
            pl.BlockSpec((1, 1, 3 * _W), lambda b, s: (0, 0, 0)),
            pl.BlockSpec((1, 1, _W), lambda b, s: (jnp.minimum(b, _NB - 1), 0, 0)),
        ],
        out_specs=[
            pl.BlockSpec((_B, _W), lambda b, s: (0, b)),
            pl.BlockSpec((_DIM, _W), lambda b, s: (0, jnp.minimum(b, _NB - 1))),
            pl.BlockSpec((1, 1, _W), lambda b, s: (jnp.minimum(b, _NB - 1), 0, 0)),
        ],
    )
    return pl.pallas_call(
        _main_body,
        grid_spec=grid_spec,
        out_shape=(
            jax.ShapeDtypeStruct((_B, _R + 1), jnp.float32),
            jax.ShapeDtypeStruct((_DIM, _R), jnp.float32),
            jax.ShapeDtypeStruct((_NB, 1, _W), jnp.int32),
        ),
        compiler_params=pltpu.CompilerParams(
            dimension_semantics=("arbitrary",),
        ),
        interpret=interpret,
    )(ptr_arr, qn_s, knt_ext, lpos, queue, queue, idx_ext, qidx_r)


def kernel(q, k, queue, index, queue_index, ptr, interpret=False):
    qn_s, knt, lpos = _prologue(q, k, interpret=interpret)
    ptr_c = jnp.clip(jnp.asarray(ptr, jnp.int32), 0, _R - _B)
    ptr_arr = ptr_c[None]
    rr = ptr_c % _W
    knt_ext = jnp.roll(jnp.pad(knt, ((0, 0), (_W, 2 * _W - _B))), rr, axis=1)
    idx_ext = jnp.roll(jnp.pad(index, (_W, 2 * _W - _B)), rr)[None, None, :]
    qidx_r = queue_index.reshape(_NB, 1, _W)
    logits, nq, nqi = _main(ptr_arr, qn_s, knt_ext, lpos, queue, idx_ext,
                            qidx_r, interpret=interpret)
    return logits, nq, nqi.reshape(_R)
